# Initial kernel scaffold; baseline (speedup 1.0000x reference)
import jax
import jax.numpy as jnp
from jax.experimental import pallas as pl


def kernel(x, edge_index, cache_name, W, b):
    raise NotImplementedError("write your pallas kernel here")


# speedup vs baseline: 259.8210x; 259.8210x over previous
import functools

import numpy as np

import jax
import jax.numpy as jnp
from jax import lax
from jax.experimental import pallas as pl
from jax.experimental.pallas import tpu as pltpu
from jax.experimental.pallas import tpu_sc as plsc

N = 10000
D = 128
E = 320000

NCORES = 2
NSUB = 16
CHUNK = 128
G = 10
NGROUPS = E // (CHUNK * G)
GROUPS_PER_CORE = NGROUPS // NCORES
ITERS = (GROUPS_PER_CORE + NSUB - 1) // NSUB

ROWS_A = 624
ROWS_B = N - 15 * ROWS_A


def _sc_mesh():
    return plsc.VectorSubcoreMesh(core_axis_name="c", subcore_axis_name="s")


def _deg_counts(dst3d, zn):
    @functools.partial(
        pl.kernel,
        out_type=[jax.ShapeDtypeStruct((N,), jnp.float32),
                  jax.ShapeDtypeStruct((N,), jnp.float32)],
        mesh=_sc_mesh(),
        scratch_types=[
            pltpu.VMEM((G, CHUNK), jnp.int32),
            pltpu.VMEM((CHUNK,), jnp.float32),
            pltpu.VMEM_SHARED((N,), jnp.float32),
        ],
    )
    def k(dst_hbm, zn_hbm, out0_hbm, out1_hbm, idx_v, ones_v, deg_sh):
        c = lax.axis_index("c")
        s = lax.axis_index("s")
        for j in range(CHUNK // 16):
            ones_v[pl.ds(j * 16, 16)] = jnp.ones((16,), jnp.float32)

        @pl.when(s == 0)
        def _():
            pltpu.sync_copy(zn_hbm, deg_sh)

        plsc.subcore_barrier()

        def body(i, carry):
            g = s + i * jnp.int32(NSUB)

            @pl.when(g < jnp.int32(GROUPS_PER_CORE))
            def _():
                r = c * jnp.int32(GROUPS_PER_CORE) + g
                pltpu.sync_copy(dst_hbm.at[r], idx_v)
                for j in range(G):
                    pltpu.sync_copy(ones_v, deg_sh.at[idx_v.at[jnp.int32(j)]],
                                    add=True)

            return carry

        lax.fori_loop(jnp.int32(0), jnp.int32(ITERS), body, None)
        plsc.subcore_barrier()

        @pl.when((s == 0) & (c == 0))
        def _():
            pltpu.sync_copy(deg_sh, out0_hbm)

        @pl.when((s == 0) & (c == 1))
        def _():
            pltpu.sync_copy(deg_sh, out1_hbm)

    return k(dst3d, zn)


def _edge_accumulate(src3d, dst3d, h2, znd):
    @functools.partial(
        pl.kernel,
        out_type=jax.ShapeDtypeStruct((NCORES, N, D), jnp.float32),
        mesh=_sc_mesh(),
        scratch_types=[
            pltpu.VMEM((G, CHUNK), jnp.int32),
            pltpu.VMEM((G, CHUNK), jnp.int32),
            pltpu.VMEM((CHUNK, D), jnp.float32),
            pltpu.VMEM_SHARED((N, D), jnp.float32),
            pltpu.SemaphoreType.DMA,
        ],
    )
    def k(src_hbm, dst_hbm, h2_hbm, znd_hbm, out_hbm, isrc_v, idst_v, rows_v,
          acc_sh, sem):
        c = lax.axis_index("c")
        s = lax.axis_index("s")

        @pl.when(s < 15)
        def _():
            r0 = s * jnp.int32(ROWS_A)
            pltpu.sync_copy(znd_hbm.at[pl.ds(r0, ROWS_A)],
                            acc_sh.at[pl.ds(r0, ROWS_A)])

        @pl.when(s == 15)
        def _():
            r0 = jnp.int32(15 * ROWS_A)
            pltpu.sync_copy(znd_hbm.at[pl.ds(r0, ROWS_B)],
                            acc_sh.at[pl.ds(r0, ROWS_B)])

        plsc.subcore_barrier()

        def body(i, carry):
            g = s + i * jnp.int32(NSUB)

            @pl.when(g < jnp.int32(GROUPS_PER_CORE))
            def _():
                r = c * jnp.int32(GROUPS_PER_CORE) + g
                pltpu.sync_copy(src_hbm.at[r], isrc_v)
                pltpu.sync_copy(dst_hbm.at[r], idst_v)
                for j in range(G):
                    j32 = jnp.int32(j)
                    pltpu.async_copy(h2_hbm.at[isrc_v.at[j32]], rows_v,
                                     sem).wait()
                    pltpu.sync_copy(rows_v, acc_sh.at[idst_v.at[j32]],
                                    add=True)

            return carry

        lax.fori_loop(jnp.int32(0), jnp.int32(ITERS), body, None)
        plsc.subcore_barrier()

        @pl.when(s < 15)
        def _():
            r0 = s * jnp.int32(ROWS_A)
            pltpu.sync_copy(acc_sh.at[pl.ds(r0, ROWS_A)],
                            out_hbm.at[c, pl.ds(r0, ROWS_A)])

        @pl.when(s == 15)
        def _():
            r0 = jnp.int32(15 * ROWS_A)
            pltpu.sync_copy(acc_sh.at[pl.ds(r0, ROWS_B)],
                            out_hbm.at[c, pl.ds(r0, ROWS_B)])

    return k(src3d, dst3d, h2, znd)


_I0 = np.int32(0)
_R = 1000


def _tc_prep(x, W, dT):
    def body(x_ref, w_ref, d_ref, h2_ref):
        dis = lax.rsqrt(1.0 + d_ref[:, 0:1] + d_ref[:, 1:2])
        h = jnp.dot(x_ref[...], w_ref[...], preferred_element_type=jnp.float32)
        h2_ref[...] = dis * h

    return pl.pallas_call(
        body,
        grid=(N // _R,),
        in_specs=[
            pl.BlockSpec((_R, D), lambda i: (i, _I0)),
            pl.BlockSpec((D, D), lambda i: (_I0, _I0)),
            pl.BlockSpec((_R, 2), lambda i: (i, _I0)),
        ],
        out_specs=pl.BlockSpec((_R, D), lambda i: (i, _I0)),
        out_shape=jax.ShapeDtypeStruct((N, D), jnp.float32),
    )(x, W, dT)


def _tc_final(accp, h2, dT, b2):
    def body(a_ref, h2_ref, d_ref, b_ref, o_ref):
        dis = lax.rsqrt(1.0 + d_ref[:, 0:1] + d_ref[:, 1:2])
        tot = a_ref[0] + a_ref[1] + h2_ref[...]
        o_ref[...] = jnp.maximum(dis * tot + b_ref[...], 0.0)

    return pl.pallas_call(
        body,
        grid=(N // _R,),
        in_specs=[
            pl.BlockSpec((NCORES, _R, D), lambda i: (_I0, i, _I0)),
            pl.BlockSpec((_R, D), lambda i: (i, _I0)),
            pl.BlockSpec((_R, 2), lambda i: (i, _I0)),
            pl.BlockSpec((1, D), lambda i: (_I0, _I0)),
        ],
        out_specs=pl.BlockSpec((_R, D), lambda i: (i, _I0)),
        out_shape=jax.ShapeDtypeStruct((N, D), jnp.float32),
    )(accp, h2, dT, b2)


def kernel(x, edge_index, cache_name, W, b):
    ei = edge_index.astype(jnp.int32)
    src3d = ei[0].reshape(NGROUPS, G, CHUNK)
    dst3d = ei[1].reshape(NGROUPS, G, CHUNK)
    zn = jnp.zeros((N,), jnp.float32)
    znd = jnp.zeros((N, D), jnp.float32)

    d0, d1 = _deg_counts(dst3d, zn)
    dT = jnp.stack([d0, d1], axis=1)
    h2 = _tc_prep(x.astype(jnp.float32), W.astype(jnp.float32), dT)
    accp = _edge_accumulate(src3d, dst3d, h2, znd)
    out = _tc_final(accp, h2, dT, b.reshape(1, D).astype(jnp.float32))
    return out.astype(W.dtype)


# speedup vs baseline: 296.2838x; 1.1403x over previous
import functools

import numpy as np

import jax
import jax.numpy as jnp
from jax import lax
from jax.experimental import pallas as pl
from jax.experimental.pallas import tpu as pltpu
from jax.experimental.pallas import tpu_sc as plsc

N = 10000
D = 128
E = 320000

NCORES = 2
NSUB = 16
CHUNK = 128
G = 10
NGROUPS = E // (CHUNK * G)
GROUPS_PER_CORE = NGROUPS // NCORES
ITERS = (GROUPS_PER_CORE + NSUB - 1) // NSUB

ROWS_A = 624
ROWS_B = N - 15 * ROWS_A


def _sc_mesh():
    return plsc.VectorSubcoreMesh(core_axis_name="c", subcore_axis_name="s")


def _deg_counts(dst3d, zn):
    @functools.partial(
        pl.kernel,
        out_type=[jax.ShapeDtypeStruct((N,), jnp.float32),
                  jax.ShapeDtypeStruct((N,), jnp.float32)],
        mesh=_sc_mesh(),
        scratch_types=[
            pltpu.VMEM((G, CHUNK), jnp.int32),
            pltpu.VMEM((CHUNK,), jnp.float32),
            pltpu.VMEM_SHARED((N,), jnp.float32),
        ],
    )
    def k(dst_hbm, zn_hbm, out0_hbm, out1_hbm, idx_v, ones_v, deg_sh):
        c = lax.axis_index("c")
        s = lax.axis_index("s")
        for j in range(CHUNK // 16):
            ones_v[pl.ds(j * 16, 16)] = jnp.ones((16,), jnp.float32)

        @pl.when(s == 0)
        def _():
            pltpu.sync_copy(zn_hbm, deg_sh)

        plsc.subcore_barrier()

        def body(i, carry):
            g = s + i * jnp.int32(NSUB)

            @pl.when(g < jnp.int32(GROUPS_PER_CORE))
            def _():
                r = c * jnp.int32(GROUPS_PER_CORE) + g
                pltpu.sync_copy(dst_hbm.at[r], idx_v)
                for j in range(G):
                    pltpu.sync_copy(ones_v, deg_sh.at[idx_v.at[jnp.int32(j)]],
                                    add=True)

            return carry

        lax.fori_loop(jnp.int32(0), jnp.int32(ITERS), body, None)
        plsc.subcore_barrier()

        @pl.when((s == 0) & (c == 0))
        def _():
            pltpu.sync_copy(deg_sh, out0_hbm)

        @pl.when((s == 0) & (c == 1))
        def _():
            pltpu.sync_copy(deg_sh, out1_hbm)

    return k(dst3d, zn)


def _edge_accumulate(src3d, dst3d, h2, znd):
    @functools.partial(
        pl.kernel,
        out_type=jax.ShapeDtypeStruct((NCORES, N, D), jnp.float32),
        mesh=_sc_mesh(),
        scratch_types=[
            pltpu.VMEM((G, CHUNK), jnp.int32),
            pltpu.VMEM((G, CHUNK), jnp.int32),
            pltpu.VMEM((CHUNK, D), jnp.float32),
            pltpu.VMEM((CHUNK, D), jnp.float32),
            pltpu.VMEM_SHARED((N, D), jnp.float32),
            pltpu.SemaphoreType.DMA,
            pltpu.SemaphoreType.DMA,
        ],
    )
    def k(src_hbm, dst_hbm, h2_hbm, znd_hbm, out_hbm, isrc_v, idst_v, rows_a,
          rows_b, acc_sh, sem_a, sem_b):
        c = lax.axis_index("c")
        s = lax.axis_index("s")

        @pl.when(s < 15)
        def _():
            r0 = s * jnp.int32(ROWS_A)
            pltpu.sync_copy(znd_hbm.at[pl.ds(r0, ROWS_A)],
                            acc_sh.at[pl.ds(r0, ROWS_A)])

        @pl.when(s == 15)
        def _():
            r0 = jnp.int32(15 * ROWS_A)
            pltpu.sync_copy(znd_hbm.at[pl.ds(r0, ROWS_B)],
                            acc_sh.at[pl.ds(r0, ROWS_B)])

        plsc.subcore_barrier()

        def body(i, carry):
            g = s + i * jnp.int32(NSUB)

            @pl.when(g < jnp.int32(GROUPS_PER_CORE))
            def _():
                r = c * jnp.int32(GROUPS_PER_CORE) + g
                pltpu.sync_copy(src_hbm.at[r], isrc_v)
                pltpu.sync_copy(dst_hbm.at[r], idst_v)
                bufs = (rows_a, rows_b)
                sems = (sem_a, sem_b)
                pltpu.async_copy(h2_hbm.at[isrc_v.at[jnp.int32(0)]],
                                 bufs[0], sems[0])
                for j in range(G):
                    j32 = jnp.int32(j)
                    pltpu.make_async_copy(h2_hbm.at[isrc_v.at[j32]],
                                          bufs[j % 2], sems[j % 2]).wait()
                    if j + 1 < G:
                        pltpu.async_copy(
                            h2_hbm.at[isrc_v.at[jnp.int32(j + 1)]],
                            bufs[(j + 1) % 2], sems[(j + 1) % 2])
                    pltpu.sync_copy(bufs[j % 2], acc_sh.at[idst_v.at[j32]],
                                    add=True)

            return carry

        lax.fori_loop(jnp.int32(0), jnp.int32(ITERS), body, None)
        plsc.subcore_barrier()

        @pl.when(s < 15)
        def _():
            r0 = s * jnp.int32(ROWS_A)
            pltpu.sync_copy(acc_sh.at[pl.ds(r0, ROWS_A)],
                            out_hbm.at[c, pl.ds(r0, ROWS_A)])

        @pl.when(s == 15)
        def _():
            r0 = jnp.int32(15 * ROWS_A)
            pltpu.sync_copy(acc_sh.at[pl.ds(r0, ROWS_B)],
                            out_hbm.at[c, pl.ds(r0, ROWS_B)])

    return k(src3d, dst3d, h2, znd)


_I0 = np.int32(0)
_R = 1000


def _tc_prep(x, W, dT):
    def body(x_ref, w_ref, d_ref, h2_ref):
        dis = lax.rsqrt(1.0 + d_ref[:, 0:1] + d_ref[:, 1:2])
        h = jnp.dot(x_ref[...], w_ref[...], preferred_element_type=jnp.float32)
        h2_ref[...] = dis * h

    return pl.pallas_call(
        body,
        grid=(N // _R,),
        in_specs=[
            pl.BlockSpec((_R, D), lambda i: (i, _I0)),
            pl.BlockSpec((D, D), lambda i: (_I0, _I0)),
            pl.BlockSpec((_R, 2), lambda i: (i, _I0)),
        ],
        out_specs=pl.BlockSpec((_R, D), lambda i: (i, _I0)),
        out_shape=jax.ShapeDtypeStruct((N, D), jnp.float32),
    )(x, W, dT)


def _tc_final(accp, h2, dT, b2):
    def body(a_ref, h2_ref, d_ref, b_ref, o_ref):
        dis = lax.rsqrt(1.0 + d_ref[:, 0:1] + d_ref[:, 1:2])
        tot = a_ref[0] + a_ref[1] + h2_ref[...]
        o_ref[...] = jnp.maximum(dis * tot + b_ref[...], 0.0)

    return pl.pallas_call(
        body,
        grid=(N // _R,),
        in_specs=[
            pl.BlockSpec((NCORES, _R, D), lambda i: (_I0, i, _I0)),
            pl.BlockSpec((_R, D), lambda i: (i, _I0)),
            pl.BlockSpec((_R, 2), lambda i: (i, _I0)),
            pl.BlockSpec((1, D), lambda i: (_I0, _I0)),
        ],
        out_specs=pl.BlockSpec((_R, D), lambda i: (i, _I0)),
        out_shape=jax.ShapeDtypeStruct((N, D), jnp.float32),
    )(accp, h2, dT, b2)


def kernel(x, edge_index, cache_name, W, b):
    ei = edge_index.astype(jnp.int32)
    src3d = ei[0].reshape(NGROUPS, G, CHUNK)
    dst3d = ei[1].reshape(NGROUPS, G, CHUNK)
    zn = jnp.zeros((N,), jnp.float32)
    znd = jnp.zeros((N, D), jnp.float32)

    d0, d1 = _deg_counts(dst3d, zn)
    dT = jnp.stack([d0, d1], axis=1)
    h2 = _tc_prep(x.astype(jnp.float32), W.astype(jnp.float32), dT)
    accp = _edge_accumulate(src3d, dst3d, h2, znd)
    out = _tc_final(accp, h2, dT, b.reshape(1, D).astype(jnp.float32))
    return out.astype(W.dtype)


# speedup vs baseline: 316.2707x; 1.0675x over previous
import functools

import numpy as np

import jax
import jax.numpy as jnp
from jax import lax
from jax.experimental import pallas as pl
from jax.experimental.pallas import tpu as pltpu
from jax.experimental.pallas import tpu_sc as plsc

N = 10000
D = 128
E = 320000

NCORES = 2
NSUB = 16
CHUNK = 128
G = 10
NGROUPS = E // (CHUNK * G)
GROUPS_PER_CORE = NGROUPS // NCORES
ITERS = (GROUPS_PER_CORE + NSUB - 1) // NSUB

ECHUNK = 100
EG = 10
ENGROUPS = E // (ECHUNK * EG)
EGROUPS_PER_CORE = ENGROUPS // NCORES
EITERS = EGROUPS_PER_CORE // NSUB

ROWS_A = 624
ROWS_B = N - 15 * ROWS_A


def _sc_mesh():
    return plsc.VectorSubcoreMesh(core_axis_name="c", subcore_axis_name="s")


def _deg_counts(dst3d, zn):
    @functools.partial(
        pl.kernel,
        out_type=[jax.ShapeDtypeStruct((N,), jnp.float32),
                  jax.ShapeDtypeStruct((N,), jnp.float32)],
        mesh=_sc_mesh(),
        scratch_types=[
            pltpu.VMEM((G, CHUNK), jnp.int32),
            pltpu.VMEM((CHUNK,), jnp.float32),
            pltpu.VMEM_SHARED((N,), jnp.float32),
        ],
    )
    def k(dst_hbm, zn_hbm, out0_hbm, out1_hbm, idx_v, ones_v, deg_sh):
        c = lax.axis_index("c")
        s = lax.axis_index("s")
        for j in range(CHUNK // 16):
            ones_v[pl.ds(j * 16, 16)] = jnp.ones((16,), jnp.float32)

        @pl.when(s == 0)
        def _():
            pltpu.sync_copy(zn_hbm, deg_sh)

        plsc.subcore_barrier()

        def body(i, carry):
            g = s + i * jnp.int32(NSUB)

            @pl.when(g < jnp.int32(GROUPS_PER_CORE))
            def _():
                r = c * jnp.int32(GROUPS_PER_CORE) + g
                pltpu.sync_copy(dst_hbm.at[r], idx_v)
                for j in range(G):
                    pltpu.sync_copy(ones_v, deg_sh.at[idx_v.at[jnp.int32(j)]],
                                    add=True)

            return carry

        lax.fori_loop(jnp.int32(0), jnp.int32(ITERS), body, None)
        plsc.subcore_barrier()

        @pl.when((s == 0) & (c == 0))
        def _():
            pltpu.sync_copy(deg_sh, out0_hbm)

        @pl.when((s == 0) & (c == 1))
        def _():
            pltpu.sync_copy(deg_sh, out1_hbm)

    return k(dst3d, zn)


def _edge_accumulate(src3d, dst3d, h2, znd):
    @functools.partial(
        pl.kernel,
        out_type=jax.ShapeDtypeStruct((NCORES, N, D), jnp.float32),
        mesh=_sc_mesh(),
        scratch_types=[
            pltpu.VMEM((EG, ECHUNK), jnp.int32),
            pltpu.VMEM((EG, ECHUNK), jnp.int32),
            pltpu.VMEM((ECHUNK, D), jnp.float32),
            pltpu.VMEM((ECHUNK, D), jnp.float32),
            pltpu.VMEM((ECHUNK, D), jnp.float32),
            pltpu.VMEM_SHARED((N, D), jnp.float32),
            pltpu.SemaphoreType.DMA,
            pltpu.SemaphoreType.DMA,
            pltpu.SemaphoreType.DMA,
            pltpu.SemaphoreType.DMA,
            pltpu.SemaphoreType.DMA,
            pltpu.SemaphoreType.DMA,
        ],
    )
    def k(src_hbm, dst_hbm, h2_hbm, znd_hbm, out_hbm, isrc_v, idst_v, rows_a,
          rows_b, rows_c, acc_sh, gsem_a, gsem_b, gsem_c,
          ssem_a, ssem_b, ssem_c):
        c = lax.axis_index("c")
        s = lax.axis_index("s")

        @pl.when(s < 15)
        def _():
            r0 = s * jnp.int32(ROWS_A)
            pltpu.sync_copy(znd_hbm.at[pl.ds(r0, ROWS_A)],
                            acc_sh.at[pl.ds(r0, ROWS_A)])

        @pl.when(s == 15)
        def _():
            r0 = jnp.int32(15 * ROWS_A)
            pltpu.sync_copy(znd_hbm.at[pl.ds(r0, ROWS_B)],
                            acc_sh.at[pl.ds(r0, ROWS_B)])

        plsc.subcore_barrier()

        def body(i, carry):
            g = s + i * jnp.int32(NSUB)

            def _():
                r = c * jnp.int32(EGROUPS_PER_CORE) + g
                pltpu.sync_copy(src_hbm.at[r], isrc_v)
                pltpu.sync_copy(dst_hbm.at[r], idst_v)
                bufs = (rows_a, rows_b, rows_c)
                gsems = (gsem_a, gsem_b, gsem_c)
                ssems = (ssem_a, ssem_b, ssem_c)

                def gather(j):
                    return pltpu.async_copy(
                        h2_hbm.at[isrc_v.at[jnp.int32(j)]], bufs[j % 3],
                        gsems[j % 3])

                def scatter(j):
                    return pltpu.async_copy(
                        bufs[j % 3], acc_sh.at[idst_v.at[jnp.int32(j)]],
                        ssems[j % 3], add=True)

                gd = {0: gather(0), 1: gather(1)}
                sd = {}
                for j in range(EG):
                    gd[j].wait()
                    if j >= 1:
                        sd[j - 1].wait()
                    sd[j] = scatter(j)
                    if j + 2 < EG:
                        gd[j + 2] = gather(j + 2)
                sd[EG - 1].wait()

            _()
            return carry

        lax.fori_loop(jnp.int32(0), jnp.int32(EITERS), body, None)
        plsc.subcore_barrier()

        @pl.when(s < 15)
        def _():
            r0 = s * jnp.int32(ROWS_A)
            pltpu.sync_copy(acc_sh.at[pl.ds(r0, ROWS_A)],
                            out_hbm.at[c, pl.ds(r0, ROWS_A)])

        @pl.when(s == 15)
        def _():
            r0 = jnp.int32(15 * ROWS_A)
            pltpu.sync_copy(acc_sh.at[pl.ds(r0, ROWS_B)],
                            out_hbm.at[c, pl.ds(r0, ROWS_B)])

    return k(src3d, dst3d, h2, znd)


_I0 = np.int32(0)
_R = 1000


def _tc_prep(x, W, d0, d1):
    def body(x_ref, w_ref, d0_ref, d1_ref, h2_ref):
        dis = lax.rsqrt(1.0 + d0_ref[...] + d1_ref[...])
        h = jnp.dot(x_ref[...], w_ref[...], preferred_element_type=jnp.float32)
        h2_ref[...] = dis * h

    return pl.pallas_call(
        body,
        grid=(N // _R,),
        in_specs=[
            pl.BlockSpec((_R, D), lambda i: (i, _I0)),
            pl.BlockSpec((D, D), lambda i: (_I0, _I0)),
            pl.BlockSpec((_R, 1), lambda i: (i, _I0)),
            pl.BlockSpec((_R, 1), lambda i: (i, _I0)),
        ],
        out_specs=pl.BlockSpec((_R, D), lambda i: (i, _I0)),
        out_shape=jax.ShapeDtypeStruct((N, D), jnp.float32),
    )(x, W, d0, d1)


def _tc_final(accp, h2, d0, d1, b2):
    def body(a_ref, h2_ref, d0_ref, d1_ref, b_ref, o_ref):
        dis = lax.rsqrt(1.0 + d0_ref[...] + d1_ref[...])
        tot = a_ref[0] + a_ref[1] + h2_ref[...]
        o_ref[...] = jnp.maximum(dis * tot + b_ref[...], 0.0)

    return pl.pallas_call(
        body,
        grid=(N // _R,),
        in_specs=[
            pl.BlockSpec((NCORES, _R, D), lambda i: (_I0, i, _I0)),
            pl.BlockSpec((_R, D), lambda i: (i, _I0)),
            pl.BlockSpec((_R, 1), lambda i: (i, _I0)),
            pl.BlockSpec((_R, 1), lambda i: (i, _I0)),
            pl.BlockSpec((1, D), lambda i: (_I0, _I0)),
        ],
        out_specs=pl.BlockSpec((_R, D), lambda i: (i, _I0)),
        out_shape=jax.ShapeDtypeStruct((N, D), jnp.float32),
    )(accp, h2, d0, d1, b2)


def kernel(x, edge_index, cache_name, W, b):
    ei = edge_index.astype(jnp.int32)
    src3d = ei[0].reshape(ENGROUPS, EG, ECHUNK)
    dst3d = ei[1].reshape(ENGROUPS, EG, ECHUNK)
    dst3d_deg = ei[1].reshape(NGROUPS, G, CHUNK)
    zn = jnp.zeros((N,), jnp.float32)
    znd = jnp.zeros((N, D), jnp.float32)

    d0, d1 = _deg_counts(dst3d_deg, zn)
    d0 = d0.reshape(N, 1)
    d1 = d1.reshape(N, 1)
    h2 = _tc_prep(x.astype(jnp.float32), W.astype(jnp.float32), d0, d1)
    accp = _edge_accumulate(src3d, dst3d, h2, znd)
    out = _tc_final(accp, h2, d0, d1, b.reshape(1, D).astype(jnp.float32))
    return out.astype(W.dtype)


# speedup vs baseline: 328.2287x; 1.0378x over previous
import functools

import numpy as np

import jax
import jax.numpy as jnp
from jax import lax
from jax.experimental import pallas as pl
from jax.experimental.pallas import tpu as pltpu
from jax.experimental.pallas import tpu_sc as plsc

N = 10000
D = 128
E = 320000

NCORES = 2
NSUB = 16
CHUNK = 128
G = 10
NGROUPS = E // (CHUNK * G)
GROUPS_PER_CORE = NGROUPS // NCORES
ITERS = (GROUPS_PER_CORE + NSUB - 1) // NSUB

ECHUNK = 100
EG = 10
ENGROUPS = E // (ECHUNK * EG)
EGROUPS_PER_CORE = ENGROUPS // NCORES
EITERS = EGROUPS_PER_CORE // NSUB

ROWS_A = 624
ROWS_B = N - 15 * ROWS_A


def _sc_mesh():
    return plsc.VectorSubcoreMesh(core_axis_name="c", subcore_axis_name="s")


def _deg_counts(dst3d, zn):
    @functools.partial(
        pl.kernel,
        out_type=[jax.ShapeDtypeStruct((N,), jnp.float32),
                  jax.ShapeDtypeStruct((N,), jnp.float32)],
        mesh=_sc_mesh(),
        scratch_types=[
            pltpu.VMEM((G, CHUNK), jnp.int32),
            pltpu.VMEM((CHUNK,), jnp.float32),
            pltpu.VMEM_SHARED((N,), jnp.float32),
        ],
    )
    def k(dst_hbm, zn_hbm, out0_hbm, out1_hbm, idx_v, ones_v, deg_sh):
        c = lax.axis_index("c")
        s = lax.axis_index("s")
        for j in range(CHUNK // 16):
            ones_v[pl.ds(j * 16, 16)] = jnp.ones((16,), jnp.float32)

        @pl.when(s == 0)
        def _():
            pltpu.sync_copy(zn_hbm, deg_sh)

        plsc.subcore_barrier()

        def body(i, carry):
            g = s + i * jnp.int32(NSUB)

            @pl.when(g < jnp.int32(GROUPS_PER_CORE))
            def _():
                r = c * jnp.int32(GROUPS_PER_CORE) + g
                pltpu.sync_copy(dst_hbm.at[r], idx_v)
                for j in range(G):
                    pltpu.sync_copy(ones_v, deg_sh.at[idx_v.at[jnp.int32(j)]],
                                    add=True)

            return carry

        lax.fori_loop(jnp.int32(0), jnp.int32(ITERS), body, None)
        plsc.subcore_barrier()

        @pl.when((s == 0) & (c == 0))
        def _():
            pltpu.sync_copy(deg_sh, out0_hbm)

        @pl.when((s == 0) & (c == 1))
        def _():
            pltpu.sync_copy(deg_sh, out1_hbm)

    return k(dst3d, zn)


def _edge_accumulate(src3d, dst3d, h2, znd):
    @functools.partial(
        pl.kernel,
        out_type=jax.ShapeDtypeStruct((NCORES, N, D), jnp.float32),
        mesh=_sc_mesh(),
        scratch_types=[
            pltpu.VMEM((2, EG, ECHUNK), jnp.int32),
            pltpu.VMEM((2, EG, ECHUNK), jnp.int32),
            pltpu.VMEM((ECHUNK, D), jnp.float32),
            pltpu.VMEM((ECHUNK, D), jnp.float32),
            pltpu.VMEM((ECHUNK, D), jnp.float32),
            pltpu.VMEM_SHARED((N, D), jnp.float32),
            pltpu.SemaphoreType.DMA,
            pltpu.SemaphoreType.DMA,
            pltpu.SemaphoreType.DMA,
            pltpu.SemaphoreType.DMA,
            pltpu.SemaphoreType.DMA,
            pltpu.SemaphoreType.DMA,
            pltpu.SemaphoreType.DMA,
            pltpu.SemaphoreType.DMA,
        ],
    )
    def k(src_hbm, dst_hbm, h2_hbm, znd_hbm, out_hbm, isrc_v, idst_v, rows_a,
          rows_b, rows_c, acc_sh, gsem_a, gsem_b, gsem_c,
          ssem_a, ssem_b, ssem_c, isem_s, isem_d):
        c = lax.axis_index("c")
        s = lax.axis_index("s")

        @pl.when(s < 15)
        def _():
            r0 = s * jnp.int32(ROWS_A)
            pltpu.sync_copy(znd_hbm.at[pl.ds(r0, ROWS_A)],
                            acc_sh.at[pl.ds(r0, ROWS_A)])

        @pl.when(s == 15)
        def _():
            r0 = jnp.int32(15 * ROWS_A)
            pltpu.sync_copy(znd_hbm.at[pl.ds(r0, ROWS_B)],
                            acc_sh.at[pl.ds(r0, ROWS_B)])

        plsc.subcore_barrier()

        def grp_row(i):
            return c * jnp.int32(EGROUPS_PER_CORE) + s + i * jnp.int32(NSUB)

        def idx_fetch(i, p):
            r = grp_row(i)
            pltpu.async_copy(src_hbm.at[r], isrc_v.at[p], isem_s)
            pltpu.async_copy(dst_hbm.at[r], idst_v.at[p], isem_d)

        def idx_wait(p):
            pltpu.make_async_copy(src_hbm.at[jnp.int32(0)], isrc_v.at[p],
                                  isem_s).wait()
            pltpu.make_async_copy(dst_hbm.at[jnp.int32(0)], idst_v.at[p],
                                  isem_d).wait()

        idx_fetch(jnp.int32(0), jnp.int32(0))

        def body(i, carry):
            p = lax.rem(i, jnp.int32(2))
            idx_wait(p)

            @pl.when(i + 1 < jnp.int32(EITERS))
            def _():
                idx_fetch(i + 1, jnp.int32(1) - p)

            bufs = (rows_a, rows_b, rows_c)
            gsems = (gsem_a, gsem_b, gsem_c)
            ssems = (ssem_a, ssem_b, ssem_c)

            def gather(j):
                return pltpu.async_copy(
                    h2_hbm.at[isrc_v.at[p, jnp.int32(j)]], bufs[j % 3],
                    gsems[j % 3])

            def scatter(j):
                return pltpu.async_copy(
                    bufs[j % 3], acc_sh.at[idst_v.at[p, jnp.int32(j)]],
                    ssems[j % 3], add=True)

            gd = {0: gather(0), 1: gather(1)}
            sd = {}
            for j in range(EG):
                gd[j].wait()
                if j >= 1:
                    sd[j - 1].wait()
                sd[j] = scatter(j)
                if j + 2 < EG:
                    gd[j + 2] = gather(j + 2)
            sd[EG - 1].wait()
            return carry

        lax.fori_loop(jnp.int32(0), jnp.int32(EITERS), body, None)
        plsc.subcore_barrier()

        @pl.when(s < 15)
        def _():
            r0 = s * jnp.int32(ROWS_A)
            pltpu.sync_copy(acc_sh.at[pl.ds(r0, ROWS_A)],
                            out_hbm.at[c, pl.ds(r0, ROWS_A)])

        @pl.when(s == 15)
        def _():
            r0 = jnp.int32(15 * ROWS_A)
            pltpu.sync_copy(acc_sh.at[pl.ds(r0, ROWS_B)],
                            out_hbm.at[c, pl.ds(r0, ROWS_B)])

    return k(src3d, dst3d, h2, znd)


_I0 = np.int32(0)
_R = 1000


def _tc_prep(x, W, d0, d1):
    def body(x_ref, w_ref, d0_ref, d1_ref, h2_ref):
        dis = lax.rsqrt(1.0 + d0_ref[...] + d1_ref[...])
        h = jnp.dot(x_ref[...], w_ref[...], preferred_element_type=jnp.float32)
        h2_ref[...] = dis * h

    return pl.pallas_call(
        body,
        grid=(N // _R,),
        in_specs=[
            pl.BlockSpec((_R, D), lambda i: (i, _I0)),
            pl.BlockSpec((D, D), lambda i: (_I0, _I0)),
            pl.BlockSpec((_R, 1), lambda i: (i, _I0)),
            pl.BlockSpec((_R, 1), lambda i: (i, _I0)),
        ],
        out_specs=pl.BlockSpec((_R, D), lambda i: (i, _I0)),
        out_shape=jax.ShapeDtypeStruct((N, D), jnp.float32),
    )(x, W, d0, d1)


def _tc_final(accp, h2, d0, d1, b2):
    def body(a_ref, h2_ref, d0_ref, d1_ref, b_ref, o_ref):
        dis = lax.rsqrt(1.0 + d0_ref[...] + d1_ref[...])
        tot = a_ref[0] + a_ref[1] + h2_ref[...]
        o_ref[...] = jnp.maximum(dis * tot + b_ref[...], 0.0)

    return pl.pallas_call(
        body,
        grid=(N // _R,),
        in_specs=[
            pl.BlockSpec((NCORES, _R, D), lambda i: (_I0, i, _I0)),
            pl.BlockSpec((_R, D), lambda i: (i, _I0)),
            pl.BlockSpec((_R, 1), lambda i: (i, _I0)),
            pl.BlockSpec((_R, 1), lambda i: (i, _I0)),
            pl.BlockSpec((1, D), lambda i: (_I0, _I0)),
        ],
        out_specs=pl.BlockSpec((_R, D), lambda i: (i, _I0)),
        out_shape=jax.ShapeDtypeStruct((N, D), jnp.float32),
    )(accp, h2, d0, d1, b2)


def kernel(x, edge_index, cache_name, W, b):
    ei = edge_index.astype(jnp.int32)
    src3d = ei[0].reshape(ENGROUPS, EG, ECHUNK)
    dst3d = ei[1].reshape(ENGROUPS, EG, ECHUNK)
    dst3d_deg = ei[1].reshape(NGROUPS, G, CHUNK)
    zn = jnp.zeros((N,), jnp.float32)
    znd = jnp.zeros((N, D), jnp.float32)

    d0, d1 = _deg_counts(dst3d_deg, zn)
    d0 = d0.reshape(N, 1)
    d1 = d1.reshape(N, 1)
    h2 = _tc_prep(x.astype(jnp.float32), W.astype(jnp.float32), d0, d1)
    accp = _edge_accumulate(src3d, dst3d, h2, znd)
    out = _tc_final(accp, h2, d0, d1, b.reshape(1, D).astype(jnp.float32))
    return out.astype(W.dtype)


# speedup vs baseline: 359.8061x; 1.0962x over previous
import functools

import numpy as np

import jax
import jax.numpy as jnp
from jax import lax
from jax.experimental import pallas as pl
from jax.experimental.pallas import tpu as pltpu
from jax.experimental.pallas import tpu_sc as plsc

N = 10000
D = 128
E = 320000

NCORES = 2
NSUB = 16
CHUNK = 128
G = 10
NGROUPS = E // (CHUNK * G)
GROUPS_PER_CORE = NGROUPS // NCORES
ITERS = (GROUPS_PER_CORE + NSUB - 1) // NSUB

ECHUNK = 80
EG = 10
ENGROUPS = E // (ECHUNK * EG)
EGROUPS_PER_CORE = ENGROUPS // NCORES
EITERS = (EGROUPS_PER_CORE + NSUB - 1) // NSUB

ROWS_A = 624
ROWS_B = N - 15 * ROWS_A


def _sc_mesh():
    return plsc.VectorSubcoreMesh(core_axis_name="c", subcore_axis_name="s")


def _deg_counts(dst3d):
    @functools.partial(
        pl.kernel,
        out_type=[jax.ShapeDtypeStruct((N,), jnp.float32),
                  jax.ShapeDtypeStruct((N,), jnp.float32)],
        mesh=_sc_mesh(),
        scratch_types=[
            pltpu.VMEM((G, CHUNK), jnp.int32),
            pltpu.VMEM((CHUNK,), jnp.float32),
            pltpu.VMEM((ROWS_B,), jnp.float32),
            pltpu.VMEM_SHARED((N,), jnp.float32),
        ],
    )
    def k(dst_hbm, out0_hbm, out1_hbm, idx_v, ones_v, zeros_v, deg_sh):
        c = lax.axis_index("c")
        s = lax.axis_index("s")
        for j in range(CHUNK // 16):
            ones_v[pl.ds(j * 16, 16)] = jnp.ones((16,), jnp.float32)

        def zb(t, carry):
            zeros_v[pl.ds(t * jnp.int32(16), 16)] = jnp.zeros((16,),
                                                              jnp.float32)
            return carry

        lax.fori_loop(jnp.int32(0), jnp.int32(ROWS_B // 16), zb, None)

        @pl.when(s < 15)
        def _():
            pltpu.sync_copy(zeros_v.at[pl.ds(0, ROWS_A)],
                            deg_sh.at[pl.ds(s * jnp.int32(ROWS_A), ROWS_A)])

        @pl.when(s == 15)
        def _():
            pltpu.sync_copy(zeros_v,
                            deg_sh.at[pl.ds(jnp.int32(15 * ROWS_A), ROWS_B)])

        plsc.subcore_barrier()

        def body(i, carry):
            g = s + i * jnp.int32(NSUB)

            @pl.when(g < jnp.int32(GROUPS_PER_CORE))
            def _():
                r = c * jnp.int32(GROUPS_PER_CORE) + g
                pltpu.sync_copy(dst_hbm.at[r], idx_v)
                for j in range(G):
                    pltpu.sync_copy(ones_v, deg_sh.at[idx_v.at[jnp.int32(j)]],
                                    add=True)

            return carry

        lax.fori_loop(jnp.int32(0), jnp.int32(ITERS), body, None)
        plsc.subcore_barrier()

        @pl.when((s == 0) & (c == 0))
        def _():
            pltpu.sync_copy(deg_sh, out0_hbm)

        @pl.when((s == 0) & (c == 1))
        def _():
            pltpu.sync_copy(deg_sh, out1_hbm)

    return k(dst3d)


def _edge_accumulate(src3d, dst3d, h2):
    @functools.partial(
        pl.kernel,
        out_type=jax.ShapeDtypeStruct((NCORES, N, D), jnp.float32),
        mesh=_sc_mesh(),
        scratch_types=[
            pltpu.VMEM((2, EG, ECHUNK), jnp.int32),
            pltpu.VMEM((2, EG, ECHUNK), jnp.int32),
            pltpu.VMEM((ECHUNK, D), jnp.float32),
            pltpu.VMEM((ECHUNK, D), jnp.float32),
            pltpu.VMEM((ECHUNK, D), jnp.float32),
            pltpu.VMEM((ECHUNK, D), jnp.float32),
            pltpu.VMEM_SHARED((N, D), jnp.float32),
            pltpu.SemaphoreType.DMA,
            pltpu.SemaphoreType.DMA,
            pltpu.SemaphoreType.DMA,
            pltpu.SemaphoreType.DMA,
            pltpu.SemaphoreType.DMA,
            pltpu.SemaphoreType.DMA,
            pltpu.SemaphoreType.DMA,
            pltpu.SemaphoreType.DMA,
            pltpu.SemaphoreType.DMA,
            pltpu.SemaphoreType.DMA,
        ],
    )
    def k(src_hbm, dst_hbm, h2_hbm, out_hbm, isrc_v, idst_v, rows_a,
          rows_b, rows_c, rows_d, acc_sh, gsem_a, gsem_b, gsem_c, gsem_d,
          ssem_a, ssem_b, ssem_c, ssem_d, isem_s, isem_d):
        c = lax.axis_index("c")
        s = lax.axis_index("s")

        def zb(t, carry):
            rows_a[t, pl.ds(0, 16)] = jnp.zeros((16,), jnp.float32)
            for q in range(1, D // 16):
                rows_a[t, pl.ds(q * 16, 16)] = jnp.zeros((16,), jnp.float32)
            return carry

        lax.fori_loop(jnp.int32(0), jnp.int32(ECHUNK), zb, None)

        @pl.when(s < 15)
        def _():
            r0 = s * jnp.int32(ROWS_A)
            for m in range(ROWS_A // ECHUNK):
                pltpu.sync_copy(
                    rows_a,
                    acc_sh.at[pl.ds(r0 + jnp.int32(m * ECHUNK), ECHUNK)])
            pltpu.sync_copy(
                rows_a.at[pl.ds(0, ROWS_A % ECHUNK)],
                acc_sh.at[pl.ds(r0 + jnp.int32(ROWS_A - ROWS_A % ECHUNK),
                                ROWS_A % ECHUNK)])

        @pl.when(s == 15)
        def _():
            r0 = jnp.int32(15 * ROWS_A)
            for m in range(ROWS_B // ECHUNK):
                pltpu.sync_copy(
                    rows_a,
                    acc_sh.at[pl.ds(r0 + jnp.int32(m * ECHUNK), ECHUNK)])

        plsc.subcore_barrier()

        def idx_fetch(i, p):
            r = c * jnp.int32(EGROUPS_PER_CORE) + s + i * jnp.int32(NSUB)
            pltpu.async_copy(src_hbm.at[r], isrc_v.at[p], isem_s)
            pltpu.async_copy(dst_hbm.at[r], idst_v.at[p], isem_d)

        def idx_wait(p):
            pltpu.make_async_copy(src_hbm.at[jnp.int32(0)], isrc_v.at[p],
                                  isem_s).wait()
            pltpu.make_async_copy(dst_hbm.at[jnp.int32(0)], idst_v.at[p],
                                  isem_d).wait()

        idx_fetch(jnp.int32(0), jnp.int32(0))

        def body(i, carry):
            p = lax.rem(i, jnp.int32(2))
            g = s + i * jnp.int32(NSUB)

            @pl.when(g < jnp.int32(EGROUPS_PER_CORE))
            def _():
                idx_wait(p)

                @pl.when(g + jnp.int32(NSUB) < jnp.int32(EGROUPS_PER_CORE))
                def _():
                    idx_fetch(i + 1, jnp.int32(1) - p)

                bufs = (rows_a, rows_b, rows_c, rows_d)
                gsems = (gsem_a, gsem_b, gsem_c, gsem_d)
                ssems = (ssem_a, ssem_b, ssem_c, ssem_d)

                def gather(j):
                    return pltpu.async_copy(
                        h2_hbm.at[isrc_v.at[p, jnp.int32(j)]], bufs[j % 4],
                        gsems[j % 4])

                def scatter(j):
                    return pltpu.async_copy(
                        bufs[j % 4], acc_sh.at[idst_v.at[p, jnp.int32(j)]],
                        ssems[j % 4], add=True)

                gd = {0: gather(0), 1: gather(1)}
                sd = {}
                for j in range(EG):
                    gd[j].wait()
                    if j >= 2:
                        sd[j - 2].wait()
                    sd[j] = scatter(j)
                    if j + 2 < EG:
                        gd[j + 2] = gather(j + 2)
                sd[EG - 2].wait()
                sd[EG - 1].wait()

            return carry

        lax.fori_loop(jnp.int32(0), jnp.int32(EITERS), body, None)
        plsc.subcore_barrier()

        @pl.when(s < 15)
        def _():
            r0 = s * jnp.int32(ROWS_A)
            pltpu.sync_copy(acc_sh.at[pl.ds(r0, ROWS_A)],
                            out_hbm.at[c, pl.ds(r0, ROWS_A)])

        @pl.when(s == 15)
        def _():
            r0 = jnp.int32(15 * ROWS_A)
            pltpu.sync_copy(acc_sh.at[pl.ds(r0, ROWS_B)],
                            out_hbm.at[c, pl.ds(r0, ROWS_B)])

    return k(src3d, dst3d, h2)


_I0 = np.int32(0)
_R = 1000


def _tc_prep(x, W, d0, d1):
    def body(x_ref, w_ref, d0_ref, d1_ref, h2_ref):
        dis = lax.rsqrt(1.0 + d0_ref[...] + d1_ref[...])
        h = jnp.dot(x_ref[...], w_ref[...], preferred_element_type=jnp.float32)
        h2_ref[...] = dis * h

    return pl.pallas_call(
        body,
        grid=(N // _R,),
        in_specs=[
            pl.BlockSpec((_R, D), lambda i: (i, _I0)),
            pl.BlockSpec((D, D), lambda i: (_I0, _I0)),
            pl.BlockSpec((_R, 1), lambda i: (i, _I0)),
            pl.BlockSpec((_R, 1), lambda i: (i, _I0)),
        ],
        out_specs=pl.BlockSpec((_R, D), lambda i: (i, _I0)),
        out_shape=jax.ShapeDtypeStruct((N, D), jnp.float32),
    )(x, W, d0, d1)


def _tc_final(accp, h2, d0, d1, b2):
    def body(a_ref, h2_ref, d0_ref, d1_ref, b_ref, o_ref):
        dis = lax.rsqrt(1.0 + d0_ref[...] + d1_ref[...])
        tot = a_ref[0] + a_ref[1] + h2_ref[...]
        o_ref[...] = jnp.maximum(dis * tot + b_ref[...], 0.0)

    return pl.pallas_call(
        body,
        grid=(N // _R,),
        in_specs=[
            pl.BlockSpec((NCORES, _R, D), lambda i: (_I0, i, _I0)),
            pl.BlockSpec((_R, D), lambda i: (i, _I0)),
            pl.BlockSpec((_R, 1), lambda i: (i, _I0)),
            pl.BlockSpec((_R, 1), lambda i: (i, _I0)),
            pl.BlockSpec((1, D), lambda i: (_I0, _I0)),
        ],
        out_specs=pl.BlockSpec((_R, D), lambda i: (i, _I0)),
        out_shape=jax.ShapeDtypeStruct((N, D), jnp.float32),
    )(accp, h2, d0, d1, b2)


def kernel(x, edge_index, cache_name, W, b):
    ei = edge_index.astype(jnp.int32)
    src3d = ei[0].reshape(ENGROUPS, EG, ECHUNK)
    dst3d = ei[1].reshape(ENGROUPS, EG, ECHUNK)
    dst3d_deg = ei[1].reshape(NGROUPS, G, CHUNK)

    d0, d1 = _deg_counts(dst3d_deg)
    d0 = d0.reshape(N, 1)
    d1 = d1.reshape(N, 1)
    h2 = _tc_prep(x.astype(jnp.float32), W.astype(jnp.float32), d0, d1)
    accp = _edge_accumulate(src3d, dst3d, h2)
    out = _tc_final(accp, h2, d0, d1, b.reshape(1, D).astype(jnp.float32))
    return out.astype(W.dtype)


# speedup vs baseline: 392.1701x; 1.0899x over previous
import functools

import numpy as np

import jax
import jax.numpy as jnp
from jax import lax
from jax.experimental import pallas as pl
from jax.experimental.pallas import tpu as pltpu
from jax.experimental.pallas import tpu_sc as plsc

N = 10000
D = 128
E = 320000

NCORES = 2
NSUB = 16
CHUNK = 128
G = 10
NGROUPS = E // (CHUNK * G)
GROUPS_PER_CORE = NGROUPS // NCORES
ITERS = (GROUPS_PER_CORE + NSUB - 1) // NSUB

ECHUNK = 100
EG = 10
ENGROUPS = E // (ECHUNK * EG)
EGROUPS_PER_CORE = ENGROUPS // NCORES
EITERS = (EGROUPS_PER_CORE + NSUB - 1) // NSUB

ROWS_A = 624
ROWS_B = N - 15 * ROWS_A


def _sc_mesh():
    return plsc.VectorSubcoreMesh(core_axis_name="c", subcore_axis_name="s")


def _deg_counts(dst3d):
    @functools.partial(
        pl.kernel,
        out_type=[jax.ShapeDtypeStruct((N,), jnp.float32),
                  jax.ShapeDtypeStruct((N,), jnp.float32)],
        mesh=_sc_mesh(),
        scratch_types=[
            pltpu.VMEM((G, CHUNK), jnp.int32),
            pltpu.VMEM((CHUNK,), jnp.float32),
            pltpu.VMEM((ROWS_B,), jnp.float32),
            pltpu.VMEM_SHARED((N,), jnp.float32),
        ],
    )
    def k(dst_hbm, out0_hbm, out1_hbm, idx_v, ones_v, zeros_v, deg_sh):
        c = lax.axis_index("c")
        s = lax.axis_index("s")
        for j in range(CHUNK // 16):
            ones_v[pl.ds(j * 16, 16)] = jnp.ones((16,), jnp.float32)

        def zb(t, carry):
            zeros_v[pl.ds(t * jnp.int32(16), 16)] = jnp.zeros((16,),
                                                              jnp.float32)
            return carry

        lax.fori_loop(jnp.int32(0), jnp.int32(ROWS_B // 16), zb, None)

        @pl.when(s < 15)
        def _():
            pltpu.sync_copy(zeros_v.at[pl.ds(0, ROWS_A)],
                            deg_sh.at[pl.ds(s * jnp.int32(ROWS_A), ROWS_A)])

        @pl.when(s == 15)
        def _():
            pltpu.sync_copy(zeros_v,
                            deg_sh.at[pl.ds(jnp.int32(15 * ROWS_A), ROWS_B)])

        plsc.subcore_barrier()

        def body(i, carry):
            g = s + i * jnp.int32(NSUB)

            @pl.when(g < jnp.int32(GROUPS_PER_CORE))
            def _():
                r = c * jnp.int32(GROUPS_PER_CORE) + g
                pltpu.sync_copy(dst_hbm.at[r], idx_v)
                for j in range(G):
                    pltpu.sync_copy(ones_v, deg_sh.at[idx_v.at[jnp.int32(j)]],
                                    add=True)

            return carry

        lax.fori_loop(jnp.int32(0), jnp.int32(ITERS), body, None)
        plsc.subcore_barrier()

        @pl.when((s == 0) & (c == 0))
        def _():
            pltpu.sync_copy(deg_sh, out0_hbm)

        @pl.when((s == 0) & (c == 1))
        def _():
            pltpu.sync_copy(deg_sh, out1_hbm)

    return k(dst3d)


def _edge_accumulate(src3d, dst3d, h2):
    @functools.partial(
        pl.kernel,
        out_type=jax.ShapeDtypeStruct((NCORES, N, D), jnp.float32),
        mesh=_sc_mesh(),
        scratch_types=[
            pltpu.VMEM((2, EG, ECHUNK), jnp.int32),
            pltpu.VMEM((2, EG, ECHUNK), jnp.int32),
            pltpu.VMEM((ECHUNK, D), jnp.float32),
            pltpu.VMEM((ECHUNK, D), jnp.float32),
            pltpu.VMEM((ECHUNK, D), jnp.float32),
            pltpu.VMEM_SHARED((N, D), jnp.float32),
            pltpu.SemaphoreType.DMA,
            pltpu.SemaphoreType.DMA,
            pltpu.SemaphoreType.DMA,
            pltpu.SemaphoreType.DMA,
            pltpu.SemaphoreType.DMA,
            pltpu.SemaphoreType.DMA,
            pltpu.SemaphoreType.DMA,
            pltpu.SemaphoreType.DMA,
        ],
    )
    def k(src_hbm, dst_hbm, h2_hbm, out_hbm, isrc_v, idst_v, rows_a,
          rows_b, rows_c, acc_sh, gsem_a, gsem_b, gsem_c,
          ssem_a, ssem_b, ssem_c, isem_s, isem_d):
        c = lax.axis_index("c")
        s = lax.axis_index("s")

        def zb(t, carry):
            rows_a[t, pl.ds(0, 16)] = jnp.zeros((16,), jnp.float32)
            for q in range(1, D // 16):
                rows_a[t, pl.ds(q * 16, 16)] = jnp.zeros((16,), jnp.float32)
            return carry

        lax.fori_loop(jnp.int32(0), jnp.int32(ECHUNK), zb, None)

        @pl.when(s < 15)
        def _():
            r0 = s * jnp.int32(ROWS_A)
            for m in range(ROWS_A // ECHUNK):
                pltpu.sync_copy(
                    rows_a,
                    acc_sh.at[pl.ds(r0 + jnp.int32(m * ECHUNK), ECHUNK)])
            pltpu.sync_copy(
                rows_a.at[pl.ds(0, ROWS_A % ECHUNK)],
                acc_sh.at[pl.ds(r0 + jnp.int32(ROWS_A - ROWS_A % ECHUNK),
                                ROWS_A % ECHUNK)])

        @pl.when(s == 15)
        def _():
            r0 = jnp.int32(15 * ROWS_A)
            for m in range(ROWS_B // ECHUNK):
                pltpu.sync_copy(
                    rows_a,
                    acc_sh.at[pl.ds(r0 + jnp.int32(m * ECHUNK), ECHUNK)])
            pltpu.sync_copy(
                rows_a.at[pl.ds(0, ROWS_B % ECHUNK)],
                acc_sh.at[pl.ds(r0 + jnp.int32(ROWS_B - ROWS_B % ECHUNK),
                                ROWS_B % ECHUNK)])

        plsc.subcore_barrier()

        def idx_fetch(i, p):
            r = c * jnp.int32(EGROUPS_PER_CORE) + s + i * jnp.int32(NSUB)
            pltpu.async_copy(src_hbm.at[r], isrc_v.at[p], isem_s)
            pltpu.async_copy(dst_hbm.at[r], idst_v.at[p], isem_d)

        def idx_wait(p):
            pltpu.make_async_copy(src_hbm.at[jnp.int32(0)], isrc_v.at[p],
                                  isem_s).wait()
            pltpu.make_async_copy(dst_hbm.at[jnp.int32(0)], idst_v.at[p],
                                  isem_d).wait()

        idx_fetch(jnp.int32(0), jnp.int32(0))

        def body(i, carry):
            p = lax.rem(i, jnp.int32(2))
            g = s + i * jnp.int32(NSUB)

            @pl.when(g < jnp.int32(EGROUPS_PER_CORE))
            def _():
                idx_wait(p)

                @pl.when(g + jnp.int32(NSUB) < jnp.int32(EGROUPS_PER_CORE))
                def _():
                    idx_fetch(i + 1, jnp.int32(1) - p)

                bufs = (rows_a, rows_b, rows_c)
                gsems = (gsem_a, gsem_b, gsem_c)
                ssems = (ssem_a, ssem_b, ssem_c)

                def gather(j):
                    return pltpu.async_copy(
                        h2_hbm.at[isrc_v.at[p, jnp.int32(j)]], bufs[j % 3],
                        gsems[j % 3])

                def scatter(j):
                    return pltpu.async_copy(
                        bufs[j % 3], acc_sh.at[idst_v.at[p, jnp.int32(j)]],
                        ssems[j % 3], add=True)

                gd = {0: gather(0), 1: gather(1)}
                sd = {}
                for j in range(EG):
                    gd[j].wait()
                    if j >= 1:
                        sd[j - 1].wait()
                    sd[j] = scatter(j)
                    if j + 2 < EG:
                        gd[j + 2] = gather(j + 2)
                sd[EG - 1].wait()

            return carry

        lax.fori_loop(jnp.int32(0), jnp.int32(EITERS), body, None)
        plsc.subcore_barrier()

        @pl.when(s < 15)
        def _():
            r0 = s * jnp.int32(ROWS_A)
            pltpu.sync_copy(acc_sh.at[pl.ds(r0, ROWS_A)],
                            out_hbm.at[c, pl.ds(r0, ROWS_A)])

        @pl.when(s == 15)
        def _():
            r0 = jnp.int32(15 * ROWS_A)
            pltpu.sync_copy(acc_sh.at[pl.ds(r0, ROWS_B)],
                            out_hbm.at[c, pl.ds(r0, ROWS_B)])

    return k(src3d, dst3d, h2)


_I0 = np.int32(0)
_R = 1000


def _tc_prep(x, W, d0, d1):
    def body(x_ref, w_ref, d0_ref, d1_ref, h2_ref):
        dis = lax.rsqrt(1.0 + d0_ref[...] + d1_ref[...])
        h = jnp.dot(x_ref[...], w_ref[...], preferred_element_type=jnp.float32)
        h2_ref[...] = dis * h

    return pl.pallas_call(
        body,
        grid=(N // _R,),
        in_specs=[
            pl.BlockSpec((_R, D), lambda i: (i, _I0)),
            pl.BlockSpec((D, D), lambda i: (_I0, _I0)),
            pl.BlockSpec((_R, 1), lambda i: (i, _I0)),
            pl.BlockSpec((_R, 1), lambda i: (i, _I0)),
        ],
        out_specs=pl.BlockSpec((_R, D), lambda i: (i, _I0)),
        out_shape=jax.ShapeDtypeStruct((N, D), jnp.float32),
    )(x, W, d0, d1)


def _tc_final(accp, h2, d0, d1, b2):
    def body(a_ref, h2_ref, d0_ref, d1_ref, b_ref, o_ref):
        dis = lax.rsqrt(1.0 + d0_ref[...] + d1_ref[...])
        tot = a_ref[0] + a_ref[1] + h2_ref[...]
        o_ref[...] = jnp.maximum(dis * tot + b_ref[...], 0.0)

    return pl.pallas_call(
        body,
        grid=(N // _R,),
        in_specs=[
            pl.BlockSpec((NCORES, _R, D), lambda i: (_I0, i, _I0)),
            pl.BlockSpec((_R, D), lambda i: (i, _I0)),
            pl.BlockSpec((_R, 1), lambda i: (i, _I0)),
            pl.BlockSpec((_R, 1), lambda i: (i, _I0)),
            pl.BlockSpec((1, D), lambda i: (_I0, _I0)),
        ],
        out_specs=pl.BlockSpec((_R, D), lambda i: (i, _I0)),
        out_shape=jax.ShapeDtypeStruct((N, D), jnp.float32),
    )(accp, h2, d0, d1, b2)


def kernel(x, edge_index, cache_name, W, b):
    ei = edge_index.astype(jnp.int32)
    src3d = ei[0].reshape(ENGROUPS, EG, ECHUNK)
    dst3d = ei[1].reshape(ENGROUPS, EG, ECHUNK)
    dst3d_deg = ei[1].reshape(NGROUPS, G, CHUNK)

    d0, d1 = _deg_counts(dst3d_deg)
    d0 = d0.reshape(N, 1)
    d1 = d1.reshape(N, 1)
    h2 = _tc_prep(x.astype(jnp.float32), W.astype(jnp.float32), d0, d1)
    accp = _edge_accumulate(src3d, dst3d, h2)
    out = _tc_final(accp, h2, d0, d1, b.reshape(1, D).astype(jnp.float32))
    return out.astype(W.dtype)


# speedup vs baseline: 407.6487x; 1.0395x over previous
import functools

import numpy as np

import jax
import jax.numpy as jnp
from jax import lax
from jax.experimental import pallas as pl
from jax.experimental.pallas import tpu as pltpu
from jax.experimental.pallas import tpu_sc as plsc

N = 10000
D = 128
E = 320000

NCORES = 2
NSUB = 16
CHUNK = 128
G = 25
NGROUPS = E // (CHUNK * G)
GROUPS_PER_CORE = NGROUPS // NCORES
ITERS = (GROUPS_PER_CORE + NSUB - 1) // NSUB

ECHUNK = 100
EG = 10
ENGROUPS = E // (ECHUNK * EG)
EGROUPS_PER_CORE = ENGROUPS // NCORES
EITERS = (EGROUPS_PER_CORE + NSUB - 1) // NSUB

ROWS_A = 624
ROWS_B = N - 15 * ROWS_A


def _sc_mesh():
    return plsc.VectorSubcoreMesh(core_axis_name="c", subcore_axis_name="s")


def _deg_counts(dst3d):
    @functools.partial(
        pl.kernel,
        out_type=[jax.ShapeDtypeStruct((N,), jnp.float32),
                  jax.ShapeDtypeStruct((N,), jnp.float32)],
        mesh=_sc_mesh(),
        scratch_types=[
            pltpu.VMEM((2, G, CHUNK), jnp.int32),
            pltpu.VMEM((CHUNK,), jnp.float32),
            pltpu.VMEM((ROWS_B,), jnp.float32),
            pltpu.VMEM_SHARED((N,), jnp.float32),
            pltpu.SemaphoreType.DMA,
            pltpu.SemaphoreType.DMA,
        ],
    )
    def k(dst_hbm, out0_hbm, out1_hbm, idx_v, ones_v, zeros_v, deg_sh,
          ssem, isem):
        c = lax.axis_index("c")
        s = lax.axis_index("s")
        for j in range(CHUNK // 16):
            ones_v[pl.ds(j * 16, 16)] = jnp.ones((16,), jnp.float32)

        def zb(t, carry):
            zeros_v[pl.ds(t * jnp.int32(16), 16)] = jnp.zeros((16,),
                                                              jnp.float32)
            return carry

        lax.fori_loop(jnp.int32(0), jnp.int32(ROWS_B // 16), zb, None)

        @pl.when(s < 15)
        def _():
            pltpu.sync_copy(zeros_v.at[pl.ds(0, ROWS_A)],
                            deg_sh.at[pl.ds(s * jnp.int32(ROWS_A), ROWS_A)])

        @pl.when(s == 15)
        def _():
            pltpu.sync_copy(zeros_v,
                            deg_sh.at[pl.ds(jnp.int32(15 * ROWS_A), ROWS_B)])

        plsc.subcore_barrier()

        def idx_fetch(i, p):
            r = c * jnp.int32(GROUPS_PER_CORE) + s + i * jnp.int32(NSUB)
            pltpu.async_copy(dst_hbm.at[r], idx_v.at[p], isem)

        def idx_wait(p):
            pltpu.make_async_copy(dst_hbm.at[jnp.int32(0)], idx_v.at[p],
                                  isem).wait()

        idx_fetch(jnp.int32(0), jnp.int32(0))

        def body(i, carry):
            p = lax.rem(i, jnp.int32(2))
            g = s + i * jnp.int32(NSUB)

            @pl.when(g < jnp.int32(GROUPS_PER_CORE))
            def _():
                idx_wait(p)

                @pl.when(g + jnp.int32(NSUB) < jnp.int32(GROUPS_PER_CORE))
                def _():
                    idx_fetch(i + 1, jnp.int32(1) - p)

                sds = [pltpu.async_copy(
                    ones_v, deg_sh.at[idx_v.at[p, jnp.int32(j)]], ssem,
                    add=True) for j in range(G)]
                for d in sds:
                    d.wait()

            return carry

        lax.fori_loop(jnp.int32(0), jnp.int32(ITERS), body, None)
        plsc.subcore_barrier()

        @pl.when((s == 0) & (c == 0))
        def _():
            pltpu.sync_copy(deg_sh, out0_hbm)

        @pl.when((s == 0) & (c == 1))
        def _():
            pltpu.sync_copy(deg_sh, out1_hbm)

    return k(dst3d)


def _edge_accumulate(src3d, dst3d, h2):
    @functools.partial(
        pl.kernel,
        out_type=jax.ShapeDtypeStruct((NCORES, N, D), jnp.float32),
        mesh=_sc_mesh(),
        scratch_types=[
            pltpu.VMEM((2, EG, ECHUNK), jnp.int32),
            pltpu.VMEM((2, EG, ECHUNK), jnp.int32),
            pltpu.VMEM((ECHUNK, D), jnp.float32),
            pltpu.VMEM((ECHUNK, D), jnp.float32),
            pltpu.VMEM((ECHUNK, D), jnp.float32),
            pltpu.VMEM_SHARED((N, D), jnp.float32),
            pltpu.SemaphoreType.DMA,
            pltpu.SemaphoreType.DMA,
            pltpu.SemaphoreType.DMA,
            pltpu.SemaphoreType.DMA,
            pltpu.SemaphoreType.DMA,
            pltpu.SemaphoreType.DMA,
            pltpu.SemaphoreType.DMA,
            pltpu.SemaphoreType.DMA,
        ],
    )
    def k(src_hbm, dst_hbm, h2_hbm, out_hbm, isrc_v, idst_v, rows_a,
          rows_b, rows_c, acc_sh, gsem_a, gsem_b, gsem_c,
          ssem_a, ssem_b, ssem_c, isem_s, isem_d):
        c = lax.axis_index("c")
        s = lax.axis_index("s")

        def zb(t, carry):
            rows_a[t, pl.ds(0, 16)] = jnp.zeros((16,), jnp.float32)
            for q in range(1, D // 16):
                rows_a[t, pl.ds(q * 16, 16)] = jnp.zeros((16,), jnp.float32)
            return carry

        lax.fori_loop(jnp.int32(0), jnp.int32(ECHUNK), zb, None)

        @pl.when(s < 15)
        def _():
            r0 = s * jnp.int32(ROWS_A)
            for m in range(ROWS_A // ECHUNK):
                pltpu.sync_copy(
                    rows_a,
                    acc_sh.at[pl.ds(r0 + jnp.int32(m * ECHUNK), ECHUNK)])
            pltpu.sync_copy(
                rows_a.at[pl.ds(0, ROWS_A % ECHUNK)],
                acc_sh.at[pl.ds(r0 + jnp.int32(ROWS_A - ROWS_A % ECHUNK),
                                ROWS_A % ECHUNK)])

        @pl.when(s == 15)
        def _():
            r0 = jnp.int32(15 * ROWS_A)
            for m in range(ROWS_B // ECHUNK):
                pltpu.sync_copy(
                    rows_a,
                    acc_sh.at[pl.ds(r0 + jnp.int32(m * ECHUNK), ECHUNK)])
            pltpu.sync_copy(
                rows_a.at[pl.ds(0, ROWS_B % ECHUNK)],
                acc_sh.at[pl.ds(r0 + jnp.int32(ROWS_B - ROWS_B % ECHUNK),
                                ROWS_B % ECHUNK)])

        plsc.subcore_barrier()

        def idx_fetch(i, p):
            r = c * jnp.int32(EGROUPS_PER_CORE) + s + i * jnp.int32(NSUB)
            pltpu.async_copy(src_hbm.at[r], isrc_v.at[p], isem_s)
            pltpu.async_copy(dst_hbm.at[r], idst_v.at[p], isem_d)

        def idx_wait(p):
            pltpu.make_async_copy(src_hbm.at[jnp.int32(0)], isrc_v.at[p],
                                  isem_s).wait()
            pltpu.make_async_copy(dst_hbm.at[jnp.int32(0)], idst_v.at[p],
                                  isem_d).wait()

        idx_fetch(jnp.int32(0), jnp.int32(0))

        def body(i, carry):
            p = lax.rem(i, jnp.int32(2))
            g = s + i * jnp.int32(NSUB)

            @pl.when(g < jnp.int32(EGROUPS_PER_CORE))
            def _():
                idx_wait(p)

                @pl.when(g + jnp.int32(NSUB) < jnp.int32(EGROUPS_PER_CORE))
                def _():
                    idx_fetch(i + 1, jnp.int32(1) - p)

                bufs = (rows_a, rows_b, rows_c)
                gsems = (gsem_a, gsem_b, gsem_c)
                ssems = (ssem_a, ssem_b, ssem_c)

                def gather(j):
                    return pltpu.async_copy(
                        h2_hbm.at[isrc_v.at[p, jnp.int32(j)]], bufs[j % 3],
                        gsems[j % 3])

                def scatter(j):
                    return pltpu.async_copy(
                        bufs[j % 3], acc_sh.at[idst_v.at[p, jnp.int32(j)]],
                        ssems[j % 3], add=True)

                gd = {0: gather(0), 1: gather(1)}
                sd = {}
                for j in range(EG):
                    gd[j].wait()
                    if j >= 1:
                        sd[j - 1].wait()
                    sd[j] = scatter(j)
                    if j + 2 < EG:
                        gd[j + 2] = gather(j + 2)
                sd[EG - 1].wait()

            return carry

        lax.fori_loop(jnp.int32(0), jnp.int32(EITERS), body, None)
        plsc.subcore_barrier()

        @pl.when(s < 15)
        def _():
            r0 = s * jnp.int32(ROWS_A)
            pltpu.sync_copy(acc_sh.at[pl.ds(r0, ROWS_A)],
                            out_hbm.at[c, pl.ds(r0, ROWS_A)])

        @pl.when(s == 15)
        def _():
            r0 = jnp.int32(15 * ROWS_A)
            pltpu.sync_copy(acc_sh.at[pl.ds(r0, ROWS_B)],
                            out_hbm.at[c, pl.ds(r0, ROWS_B)])

    return k(src3d, dst3d, h2)


_I0 = np.int32(0)
_R = 1000


def _tc_prep(x, W, d0, d1):
    def body(x_ref, w_ref, d0_ref, d1_ref, h2_ref):
        dis = lax.rsqrt(1.0 + d0_ref[...] + d1_ref[...])
        h = jnp.dot(x_ref[...], w_ref[...], preferred_element_type=jnp.float32)
        h2_ref[...] = dis * h

    return pl.pallas_call(
        body,
        grid=(N // _R,),
        in_specs=[
            pl.BlockSpec((_R, D), lambda i: (i, _I0)),
            pl.BlockSpec((D, D), lambda i: (_I0, _I0)),
            pl.BlockSpec((_R, 1), lambda i: (i, _I0)),
            pl.BlockSpec((_R, 1), lambda i: (i, _I0)),
        ],
        out_specs=pl.BlockSpec((_R, D), lambda i: (i, _I0)),
        out_shape=jax.ShapeDtypeStruct((N, D), jnp.float32),
    )(x, W, d0, d1)


def _tc_final(accp, h2, d0, d1, b2):
    def body(a_ref, h2_ref, d0_ref, d1_ref, b_ref, o_ref):
        dis = lax.rsqrt(1.0 + d0_ref[...] + d1_ref[...])
        tot = a_ref[0] + a_ref[1] + h2_ref[...]
        o_ref[...] = jnp.maximum(dis * tot + b_ref[...], 0.0)

    return pl.pallas_call(
        body,
        grid=(N // _R,),
        in_specs=[
            pl.BlockSpec((NCORES, _R, D), lambda i: (_I0, i, _I0)),
            pl.BlockSpec((_R, D), lambda i: (i, _I0)),
            pl.BlockSpec((_R, 1), lambda i: (i, _I0)),
            pl.BlockSpec((_R, 1), lambda i: (i, _I0)),
            pl.BlockSpec((1, D), lambda i: (_I0, _I0)),
        ],
        out_specs=pl.BlockSpec((_R, D), lambda i: (i, _I0)),
        out_shape=jax.ShapeDtypeStruct((N, D), jnp.float32),
    )(accp, h2, d0, d1, b2)


def kernel(x, edge_index, cache_name, W, b):
    ei = edge_index.astype(jnp.int32)
    src3d = ei[0].reshape(ENGROUPS, EG, ECHUNK)
    dst3d = ei[1].reshape(ENGROUPS, EG, ECHUNK)
    dst3d_deg = ei[1].reshape(NGROUPS, G, CHUNK)

    d0, d1 = _deg_counts(dst3d_deg)
    d0 = d0.reshape(N, 1)
    d1 = d1.reshape(N, 1)
    h2 = _tc_prep(x.astype(jnp.float32), W.astype(jnp.float32), d0, d1)
    accp = _edge_accumulate(src3d, dst3d, h2)
    out = _tc_final(accp, h2, d0, d1, b.reshape(1, D).astype(jnp.float32))
    return out.astype(W.dtype)
